# baseline (speedup 1.0000x reference)
import functools

import jax
import jax.numpy as jnp
from jax import lax
from jax.experimental import pallas as pl
from jax.experimental.pallas import tpu as pltpu
from jax.experimental.pallas import tpu_sc as plsc

N_NODES = 10000
D = 128
N_EDGES = 320000
NC = 2
NS = 16
NW = NC * NS
EPW = N_EDGES // NW
CHUNK = 80
NCHUNK = EPW // CHUNK
ROWS_PT = N_NODES // NS


def _full16(v):
  return jnp.full((16,), v, dtype=jnp.int32)


KB = 25
NB = NCHUNK // KB


def _spmm_body(src_hbm, dst_hbm, w_hbm, feat_hbm, zeros_hbm, out_hbm,
               srcb, dstb, wb, rows, xacc, gsem, ssem):
  cid = lax.axis_index("c")
  sid = lax.axis_index("s")
  wid = cid * NS + sid

  pltpu.sync_copy(zeros_hbm, xacc.at[pl.ds(sid * ROWS_PT, ROWS_PT)])

  plsc.subcore_barrier()

  def mult(k):
    p = lax.rem(k, 3)

    def grp_body(g, carry):
      wg = wb[k, pl.ds(g * 16, 16)]
      for j in range(16):
        e = g * 16 + j
        wv = wg[j]
        for q in range(D // 16):
          sl = pl.ds(q * 16, 16)
          rows[p, e, sl] = rows[p, e, sl] * wv
      return carry

    lax.fori_loop(0, CHUNK // 16, grp_body, 0, unroll=False)

  def blk_body(b, carry):
    pltpu.sync_copy(src_hbm.at[wid, b], srcb)
    pltpu.sync_copy(dst_hbm.at[wid, b], dstb)
    pltpu.sync_copy(w_hbm.at[wid, b], wb)

    pltpu.async_copy(feat_hbm.at[srcb.at[0]], rows.at[0], gsem)

    def chunk_body(k, carry2):
      p = lax.rem(k, 3)
      pn = lax.rem(k + 1, 3)
      @pl.when(k >= 2)
      def _():
        pltpu.make_async_copy(rows.at[pn], xacc.at[dstb.at[k - 2]], ssem).wait()
      pltpu.make_async_copy(feat_hbm.at[srcb.at[k]], rows.at[p], gsem).wait()
      @pl.when(k + 1 < KB)
      def _():
        pltpu.async_copy(feat_hbm.at[srcb.at[k + 1]], rows.at[pn], gsem)
      mult(k)
      pltpu.async_copy(rows.at[p], xacc.at[dstb.at[k]], ssem, add=True)
      return carry2

    lax.fori_loop(0, KB, chunk_body, 0, unroll=False)

    pltpu.make_async_copy(rows.at[(KB - 2) % 3], xacc.at[dstb.at[KB - 2]],
                          ssem).wait()
    pltpu.make_async_copy(rows.at[(KB - 1) % 3], xacc.at[dstb.at[KB - 1]],
                          ssem).wait()
    return carry

  lax.fori_loop(0, NB, blk_body, 0, unroll=False)

  plsc.subcore_barrier()

  pltpu.sync_copy(xacc.at[pl.ds(sid * ROWS_PT, ROWS_PT)], out_hbm.at[cid, sid])


@functools.partial(jax.jit, static_argnames=())
def _spmm(src, dst, w, features, zeros):
  mesh = plsc.VectorSubcoreMesh(core_axis_name="c", subcore_axis_name="s")
  k = pl.kernel(
      _spmm_body,
      out_type=jax.ShapeDtypeStruct((NC, NS, ROWS_PT, D), jnp.float32),
      mesh=mesh,
      scratch_types=[
          pltpu.VMEM((KB, CHUNK), jnp.int32),
          pltpu.VMEM((KB, CHUNK), jnp.int32),
          pltpu.VMEM((KB, CHUNK), jnp.float32),
          pltpu.VMEM((3, CHUNK, D), jnp.float32),
          pltpu.VMEM_SHARED((N_NODES, D), jnp.float32),
          pltpu.SemaphoreType.DMA,
          pltpu.SemaphoreType.DMA,
      ],
  )
  return k(src, dst, w, features, zeros)


def _dense_body(f_ref, xp_ref, w1_ref, w2_ref, b_ref, o_ref):
  x = xp_ref[0] + xp_ref[1]
  f = f_ref[...]
  o_ref[...] = (
      jnp.dot(f + x, w1_ref[...], preferred_element_type=jnp.float32)
      + jnp.dot(x * f, w2_ref[...], preferred_element_type=jnp.float32)
      + b_ref[...]
  )


def _dense(features, xp, W1, W2, b):
  blk = 1000
  grid = N_NODES // blk
  return pl.pallas_call(
      _dense_body,
      grid=(grid,),
      in_specs=[
          pl.BlockSpec((blk, D), lambda i: (i, 0)),
          pl.BlockSpec((NC, blk, D), lambda i: (0, i, 0)),
          pl.BlockSpec((D, D), lambda i: (0, 0)),
          pl.BlockSpec((D, D), lambda i: (0, 0)),
          pl.BlockSpec((1, D), lambda i: (0, 0)),
      ],
      out_specs=pl.BlockSpec((blk, D), lambda i: (i, 0)),
      out_shape=jax.ShapeDtypeStruct((N_NODES, D), jnp.float32),
  )(features, xp, W1, W2, b)


def kernel(edge_index, edge_weight, features, W1, b1, W2, b2):
  src = edge_index[1].reshape(NW, NB, KB, CHUNK)
  dst = edge_index[0].reshape(NW, NB, KB, CHUNK)
  w = edge_weight.reshape(NW, NB, KB, CHUNK)
  zeros = jnp.zeros((ROWS_PT, D), jnp.float32)
  xp = _spmm(src, dst, w, features, zeros).reshape(NC, N_NODES, D)
  b = (b1 + b2).reshape(1, D)
  return _dense(features, xp, W1, W2, b)


# speedup vs baseline: 2.0114x; 2.0114x over previous
import functools

import jax
import jax.numpy as jnp
from jax import lax
from jax.experimental import pallas as pl
from jax.experimental.pallas import tpu as pltpu
from jax.experimental.pallas import tpu_sc as plsc

N_NODES = 10000
D = 128
N_EDGES = 320000
NC = 2
NS = 16
NW = NC * NS
EPW = N_EDGES // NW
CHUNK = 80
NCHUNK = EPW // CHUNK
ROWS_PT = N_NODES // NS


def _full16(v):
  return jnp.full((16,), v, dtype=jnp.int32)


KB = 25
NB = NCHUNK // KB


def _spmm_body(src_hbm, dst_hbm, w_hbm, feat_hbm, zeros_hbm, out_hbm,
               srcb, dstb, wb, rows, xacc, gsem, ssem):
  cid = lax.axis_index("c")
  sid = lax.axis_index("s")
  wid = cid * NS + sid

  pltpu.sync_copy(zeros_hbm, xacc.at[pl.ds(sid * ROWS_PT, ROWS_PT)])

  plsc.subcore_barrier()

  def mult(k):
    p = lax.rem(k, 3)
    for g in range(CHUNK // 16):
      wg = wb[k, pl.ds(g * 16, 16)]
      for j in range(16):
        e = g * 16 + j
        wv = wg[j]
        for q in range(D // 16):
          sl = pl.ds(q * 16, 16)
          rows[p, e, sl] = rows[p, e, sl] * wv

  def blk_body(b, carry):
    pltpu.sync_copy(src_hbm.at[wid, b], srcb)
    pltpu.sync_copy(dst_hbm.at[wid, b], dstb)
    pltpu.sync_copy(w_hbm.at[wid, b], wb)

    pltpu.async_copy(feat_hbm.at[srcb.at[0]], rows.at[0], gsem)

    def chunk_body(k, carry2):
      p = lax.rem(k, 3)
      pn = lax.rem(k + 1, 3)
      @pl.when(k >= 2)
      def _():
        pltpu.make_async_copy(rows.at[pn], xacc.at[dstb.at[k - 2]], ssem).wait()
      pltpu.make_async_copy(feat_hbm.at[srcb.at[k]], rows.at[p], gsem).wait()
      @pl.when(k + 1 < KB)
      def _():
        pltpu.async_copy(feat_hbm.at[srcb.at[k + 1]], rows.at[pn], gsem)
      mult(k)
      pltpu.async_copy(rows.at[p], xacc.at[dstb.at[k]], ssem, add=True)
      return carry2

    lax.fori_loop(0, KB, chunk_body, 0, unroll=False)

    pltpu.make_async_copy(rows.at[(KB - 2) % 3], xacc.at[dstb.at[KB - 2]],
                          ssem).wait()
    pltpu.make_async_copy(rows.at[(KB - 1) % 3], xacc.at[dstb.at[KB - 1]],
                          ssem).wait()
    return carry

  lax.fori_loop(0, NB, blk_body, 0, unroll=False)

  plsc.subcore_barrier()

  pltpu.sync_copy(xacc.at[pl.ds(sid * ROWS_PT, ROWS_PT)], out_hbm.at[cid, sid])


@functools.partial(jax.jit, static_argnames=())
def _spmm(src, dst, w, features, zeros):
  mesh = plsc.VectorSubcoreMesh(core_axis_name="c", subcore_axis_name="s")
  k = pl.kernel(
      _spmm_body,
      out_type=jax.ShapeDtypeStruct((NC, NS, ROWS_PT, D), jnp.float32),
      mesh=mesh,
      scratch_types=[
          pltpu.VMEM((KB, CHUNK), jnp.int32),
          pltpu.VMEM((KB, CHUNK), jnp.int32),
          pltpu.VMEM((KB, CHUNK), jnp.float32),
          pltpu.VMEM((3, CHUNK, D), jnp.float32),
          pltpu.VMEM_SHARED((N_NODES, D), jnp.float32),
          pltpu.SemaphoreType.DMA,
          pltpu.SemaphoreType.DMA,
      ],
  )
  return k(src, dst, w, features, zeros)


def _dense_body(f_ref, xp_ref, w1_ref, w2_ref, b_ref, o_ref):
  x = xp_ref[0] + xp_ref[1]
  f = f_ref[...]
  o_ref[...] = (
      jnp.dot(f + x, w1_ref[...], preferred_element_type=jnp.float32)
      + jnp.dot(x * f, w2_ref[...], preferred_element_type=jnp.float32)
      + b_ref[...]
  )


def _dense(features, xp, W1, W2, b):
  blk = 1000
  grid = N_NODES // blk
  return pl.pallas_call(
      _dense_body,
      grid=(grid,),
      in_specs=[
          pl.BlockSpec((blk, D), lambda i: (i, 0)),
          pl.BlockSpec((NC, blk, D), lambda i: (0, i, 0)),
          pl.BlockSpec((D, D), lambda i: (0, 0)),
          pl.BlockSpec((D, D), lambda i: (0, 0)),
          pl.BlockSpec((1, D), lambda i: (0, 0)),
      ],
      out_specs=pl.BlockSpec((blk, D), lambda i: (i, 0)),
      out_shape=jax.ShapeDtypeStruct((N_NODES, D), jnp.float32),
  )(features, xp, W1, W2, b)


def kernel(edge_index, edge_weight, features, W1, b1, W2, b2):
  src = edge_index[1].reshape(NW, NB, KB, CHUNK)
  dst = edge_index[0].reshape(NW, NB, KB, CHUNK)
  w = edge_weight.reshape(NW, NB, KB, CHUNK)
  zeros = jnp.zeros((ROWS_PT, D), jnp.float32)
  xp = _spmm(src, dst, w, features, zeros).reshape(NC, N_NODES, D)
  b = (b1 + b2).reshape(1, D)
  return _dense(features, xp, W1, W2, b)


# speedup vs baseline: 2.4442x; 1.2151x over previous
import functools

import jax
import jax.numpy as jnp
from jax import lax
from jax.experimental import pallas as pl
from jax.experimental.pallas import tpu as pltpu
from jax.experimental.pallas import tpu_sc as plsc

N_NODES = 10000
D = 128
N_EDGES = 320000
NC = 2
NS = 16
NW = NC * NS
EPW = N_EDGES // NW
CHUNK = 80
NCHUNK = EPW // CHUNK
ROWS_PT = N_NODES // NS


def _full16(v):
  return jnp.full((16,), v, dtype=jnp.int32)


KB = 25
NB = NCHUNK // KB


def _spmm_body(src_hbm, dst_hbm, w_hbm, feat_hbm, zeros_hbm, out_hbm,
               srcb, dstb, wb, rows, xacc, gsem, ssem):
  cid = lax.axis_index("c")
  sid = lax.axis_index("s")
  wid = cid * NS + sid

  zinit = pltpu.async_copy(zeros_hbm, xacc.at[pl.ds(sid * ROWS_PT, ROWS_PT)],
                           ssem)

  def stage(b):
    pltpu.sync_copy(src_hbm.at[wid, b], srcb)
    pltpu.sync_copy(dst_hbm.at[wid, b], dstb)
    pltpu.sync_copy(w_hbm.at[wid, b], wb)

  stage(0)
  zinit.wait()
  plsc.subcore_barrier()

  def mult(k):
    p = lax.rem(k, 4)
    for g in range(CHUNK // 16):
      wg = wb[pl.ds(k * CHUNK + g * 16, 16)]
      for j in range(16):
        e = g * 16 + j
        wv = wg[j]
        for q in range(D // 16):
          sl = pl.ds(q * 16, 16)
          rows[p, e, sl] = rows[p, e, sl] * wv

  def blk_body(b, carry):
    pltpu.async_copy(
        feat_hbm.at[srcb.at[pl.ds(0, CHUNK)]], rows.at[0], gsem)
    pltpu.async_copy(
        feat_hbm.at[srcb.at[pl.ds(CHUNK, CHUNK)]], rows.at[1], gsem)

    def chunk_body(k, carry2):
      p = lax.rem(k, 4)
      p2 = lax.rem(k + 2, 4)
      @pl.when(k >= 2)
      def _():
        pltpu.make_async_copy(rows.at[p2], xacc.at[dstb.at[k - 2]],
                              ssem).wait()
      pltpu.make_async_copy(feat_hbm.at[srcb.at[pl.ds(k * CHUNK, CHUNK)]],
                            rows.at[p], gsem).wait()
      @pl.when(k + 2 < KB)
      def _():
        pltpu.async_copy(feat_hbm.at[srcb.at[pl.ds((k + 2) * CHUNK, CHUNK)]],
                         rows.at[p2], gsem)
      mult(k)
      pltpu.async_copy(rows.at[p], xacc.at[dstb.at[k]], ssem, add=True)
      return carry2

    lax.fori_loop(0, KB, chunk_body, 0, unroll=False)

    pltpu.make_async_copy(rows.at[(KB - 2) % 4], xacc.at[dstb.at[KB - 2]],
                          ssem).wait()
    pltpu.make_async_copy(rows.at[(KB - 1) % 4], xacc.at[dstb.at[KB - 1]],
                          ssem).wait()
    @pl.when(b + 1 < NB)
    def _():
      stage(b + 1)
    return carry

  lax.fori_loop(0, NB, blk_body, 0, unroll=False)

  plsc.subcore_barrier()

  pltpu.sync_copy(xacc.at[pl.ds(sid * ROWS_PT, ROWS_PT)], out_hbm.at[cid, sid])


@functools.partial(jax.jit, static_argnames=())
def _spmm(src, dst, w, features, zeros):
  mesh = plsc.VectorSubcoreMesh(core_axis_name="c", subcore_axis_name="s")
  k = pl.kernel(
      _spmm_body,
      out_type=jax.ShapeDtypeStruct((NC, NS, ROWS_PT, D), jnp.float32),
      mesh=mesh,
      scratch_types=[
          pltpu.VMEM((KB * CHUNK,), jnp.int32),
          pltpu.VMEM((KB, CHUNK), jnp.int32),
          pltpu.VMEM((KB * CHUNK,), jnp.float32),
          pltpu.VMEM((4, CHUNK, D), jnp.float32),
          pltpu.VMEM_SHARED((N_NODES, D), jnp.float32),
          pltpu.SemaphoreType.DMA,
          pltpu.SemaphoreType.DMA,
      ],
  )
  return k(src, dst, w, features, zeros)


def _dense_body(f_ref, xp_ref, w1_ref, w2_ref, b_ref, o_ref):
  x = xp_ref[0] + xp_ref[1]
  f = f_ref[...]
  o_ref[...] = (
      jnp.dot(f + x, w1_ref[...], preferred_element_type=jnp.float32)
      + jnp.dot(x * f, w2_ref[...], preferred_element_type=jnp.float32)
      + b_ref[...]
  )


def _dense(features, xp, W1, W2, b):
  blk = 1000
  grid = N_NODES // blk
  return pl.pallas_call(
      _dense_body,
      grid=(grid,),
      in_specs=[
          pl.BlockSpec((blk, D), lambda i: (i, 0)),
          pl.BlockSpec((NC, blk, D), lambda i: (0, i, 0)),
          pl.BlockSpec((D, D), lambda i: (0, 0)),
          pl.BlockSpec((D, D), lambda i: (0, 0)),
          pl.BlockSpec((1, D), lambda i: (0, 0)),
      ],
      out_specs=pl.BlockSpec((blk, D), lambda i: (i, 0)),
      out_shape=jax.ShapeDtypeStruct((N_NODES, D), jnp.float32),
  )(features, xp, W1, W2, b)


def kernel(edge_index, edge_weight, features, W1, b1, W2, b2):
  src = edge_index[1].reshape(NW, NB, KB * CHUNK)
  dst = edge_index[0].reshape(NW, NB, KB, CHUNK)
  w = edge_weight.reshape(NW, NB, KB * CHUNK)
  zeros = jnp.zeros((ROWS_PT, D), jnp.float32)
  xp = _spmm(src, dst, w, features, zeros).reshape(NC, N_NODES, D)
  b = (b1 + b2).reshape(1, D)
  return _dense(features, xp, W1, W2, b)


# speedup vs baseline: 2.4513x; 1.0029x over previous
import functools

import jax
import jax.numpy as jnp
from jax import lax
from jax.experimental import pallas as pl
from jax.experimental.pallas import tpu as pltpu
from jax.experimental.pallas import tpu_sc as plsc

N_NODES = 10000
D = 128
N_EDGES = 320000
NC = 2
NS = 16
NW = NC * NS
EPW = N_EDGES // NW
CHUNK = 80
NCHUNK = EPW // CHUNK
ROWS_PT = N_NODES // NS


def _full16(v):
  return jnp.full((16,), v, dtype=jnp.int32)


KB = 25
NB = NCHUNK // KB


def _spmm_body(src_hbm, dst_hbm, w_hbm, feat_hbm, zeros_hbm, out_hbm,
               srcb, dstb, wb, rows, xacc, gsem, ssem):
  cid = lax.axis_index("c")
  sid = lax.axis_index("s")
  wid = cid * NS + sid

  zinit = pltpu.async_copy(zeros_hbm, xacc.at[pl.ds(sid * ROWS_PT, ROWS_PT)],
                           ssem)

  def stage(b):
    pltpu.sync_copy(src_hbm.at[wid, b], srcb)
    pltpu.sync_copy(dst_hbm.at[wid, b], dstb)
    pltpu.sync_copy(w_hbm.at[wid, b], wb)

  stage(0)
  zinit.wait()
  plsc.subcore_barrier()

  def mult(k):
    p = lax.rem(k, 4)
    for g in range(CHUNK // 16):
      wg = wb[pl.ds(k * CHUNK + g * 16, 16)]
      for j in range(16):
        e = g * 16 + j
        wv = wg[j]
        for q in range(D // 16):
          sl = pl.ds(q * 16, 16)
          rows[p, e, sl] = rows[p, e, sl] * wv

  def blk_body(b, carry):
    for t in range(2):
      for h in range(2):
        pltpu.async_copy(
            feat_hbm.at[srcb.at[pl.ds(t * CHUNK + h * (CHUNK // 2),
                                      CHUNK // 2)]],
            rows.at[t, pl.ds(h * (CHUNK // 2), CHUNK // 2)], gsem)

    def chunk_body(k, carry2):
      p = lax.rem(k, 4)
      p2 = lax.rem(k + 2, 4)
      @pl.when(k >= 2)
      def _():
        pltpu.make_async_copy(rows.at[p2], xacc.at[dstb.at[k - 2]],
                              ssem).wait()
      for h in range(2):
        pltpu.make_async_copy(
            feat_hbm.at[srcb.at[pl.ds(k * CHUNK + h * (CHUNK // 2),
                                      CHUNK // 2)]],
            rows.at[p, pl.ds(h * (CHUNK // 2), CHUNK // 2)], gsem).wait()
      @pl.when(k + 2 < KB)
      def _():
        for h in range(2):
          pltpu.async_copy(
              feat_hbm.at[srcb.at[pl.ds((k + 2) * CHUNK + h * (CHUNK // 2),
                                        CHUNK // 2)]],
              rows.at[p2, pl.ds(h * (CHUNK // 2), CHUNK // 2)], gsem)
      mult(k)
      pltpu.async_copy(rows.at[p], xacc.at[dstb.at[k]], ssem, add=True)
      return carry2

    lax.fori_loop(0, KB, chunk_body, 0, unroll=False)

    pltpu.make_async_copy(rows.at[(KB - 2) % 4], xacc.at[dstb.at[KB - 2]],
                          ssem).wait()
    pltpu.make_async_copy(rows.at[(KB - 1) % 4], xacc.at[dstb.at[KB - 1]],
                          ssem).wait()
    @pl.when(b + 1 < NB)
    def _():
      stage(b + 1)
    return carry

  lax.fori_loop(0, NB, blk_body, 0, unroll=False)

  plsc.subcore_barrier()

  pltpu.sync_copy(xacc.at[pl.ds(sid * ROWS_PT, ROWS_PT)], out_hbm.at[cid, sid])


@functools.partial(jax.jit, static_argnames=())
def _spmm(src, dst, w, features, zeros):
  mesh = plsc.VectorSubcoreMesh(core_axis_name="c", subcore_axis_name="s")
  k = pl.kernel(
      _spmm_body,
      out_type=jax.ShapeDtypeStruct((NC, NS, ROWS_PT, D), jnp.float32),
      mesh=mesh,
      scratch_types=[
          pltpu.VMEM((KB * CHUNK,), jnp.int32),
          pltpu.VMEM((KB, CHUNK), jnp.int32),
          pltpu.VMEM((KB * CHUNK,), jnp.float32),
          pltpu.VMEM((4, CHUNK, D), jnp.float32),
          pltpu.VMEM_SHARED((N_NODES, D), jnp.float32),
          pltpu.SemaphoreType.DMA,
          pltpu.SemaphoreType.DMA,
      ],
  )
  return k(src, dst, w, features, zeros)


def _dense_body(f_ref, xp_ref, w1_ref, w2_ref, b_ref, o_ref):
  x = xp_ref[0] + xp_ref[1]
  f = f_ref[...]
  o_ref[...] = (
      jnp.dot(f + x, w1_ref[...], preferred_element_type=jnp.float32)
      + jnp.dot(x * f, w2_ref[...], preferred_element_type=jnp.float32)
      + b_ref[...]
  )


def _dense(features, xp, W1, W2, b):
  blk = 1000
  grid = N_NODES // blk
  return pl.pallas_call(
      _dense_body,
      grid=(grid,),
      in_specs=[
          pl.BlockSpec((blk, D), lambda i: (i, 0)),
          pl.BlockSpec((NC, blk, D), lambda i: (0, i, 0)),
          pl.BlockSpec((D, D), lambda i: (0, 0)),
          pl.BlockSpec((D, D), lambda i: (0, 0)),
          pl.BlockSpec((1, D), lambda i: (0, 0)),
      ],
      out_specs=pl.BlockSpec((blk, D), lambda i: (i, 0)),
      out_shape=jax.ShapeDtypeStruct((N_NODES, D), jnp.float32),
  )(features, xp, W1, W2, b)


def kernel(edge_index, edge_weight, features, W1, b1, W2, b2):
  src = edge_index[1].reshape(NW, NB, KB * CHUNK)
  dst = edge_index[0].reshape(NW, NB, KB, CHUNK)
  w = edge_weight.reshape(NW, NB, KB * CHUNK)
  zeros = jnp.zeros((ROWS_PT, D), jnp.float32)
  xp = _spmm(src, dst, w, features, zeros).reshape(NC, N_NODES, D)
  b = (b1 + b2).reshape(1, D)
  return _dense(features, xp, W1, W2, b)
